# Initial kernel scaffold; baseline (speedup 1.0000x reference)
import jax
import jax.numpy as jnp
from jax.experimental import pallas as pl


def kernel(hidden_states, wq, bq, wk, bk, wv, bv, wo, bo, gamma, beta):
    raise NotImplementedError("write your pallas kernel here")


# speedup vs baseline: 1.8176x; 1.8176x over previous
import math
from functools import partial

import jax
import jax.numpy as jnp
from jax import lax
from jax.experimental import pallas as pl
from jax.experimental.pallas import tpu as pltpu

_HIDDEN = 16
_NUM_HEADS = 2
_HEAD_DIM = _HIDDEN // _NUM_HEADS
_LN_EPS = 1e-12


def _attn_kernel(x_ref, w_ref, vec_ref, out_ref, *, G, S, H, num_heads, head_dim):
    x2 = x_ref[...].reshape(G * S, H)
    w = w_ref[...]
    vec = vec_ref[...]

    proj = jnp.dot(x2, w, preferred_element_type=jnp.float32) + vec[0:1, 0:4 * H]

    y_parts = []
    for g in range(G):
        pg = proj[g * S:(g + 1) * S, :]
        probs = []
        for h in range(num_heads):
            lo = h * head_dim
            q = pg[:, lo:lo + head_dim]
            k = pg[:, H + lo:H + lo + head_dim]
            s = lax.dot_general(q, k, (((1,), (1,)), ((), ())),
                                preferred_element_type=jnp.float32)
            s = s - jnp.max(s, axis=-1, keepdims=True)
            e = jnp.exp(s)
            probs.append(e / jnp.sum(e, axis=-1, keepdims=True))
        p_cat = jnp.concatenate(probs, axis=1)
        vo = jnp.concatenate(
            [pg[:, 2 * H + h * H:2 * H + (h + 1) * H] for h in range(num_heads)],
            axis=0)
        y_parts.append(jnp.dot(p_cat, vo, preferred_element_type=jnp.float32))

    y = jnp.concatenate(y_parts, axis=0) + x2 + vec[0:1, 4 * H:5 * H]

    mean = jnp.mean(y, axis=-1, keepdims=True)
    mean_sq = jnp.mean(y * y, axis=-1, keepdims=True)
    var = mean_sq - mean * mean
    out = (y - mean) * lax.rsqrt(var + _LN_EPS) * vec[0:1, 5 * H:6 * H] \
        + vec[0:1, 6 * H:7 * H]

    out_ref[...] = out.reshape(G, S, H).astype(out_ref.dtype)


def kernel(hidden_states, wq, bq, wk, bk, wv, bv, wo, bo, gamma, beta):
    B, S, H = hidden_states.shape
    nh = _NUM_HEADS
    hd = H // nh
    scale = 1.0 / math.sqrt(hd)

    wo_t = wo.T
    wvo = [wv.T[:, h * hd:(h + 1) * hd] @ wo_t[h * hd:(h + 1) * hd, :]
           for h in range(nh)]
    bvo = [bv[h * hd:(h + 1) * hd] @ wo_t[h * hd:(h + 1) * hd, :]
           for h in range(nh)]

    w_pack = jnp.concatenate([wq.T * scale, wk.T] + wvo, axis=1)
    vec_pack = jnp.concatenate(
        [bq * scale, bk] + bvo + [bo, gamma, beta])[None, :]

    G = next(g for g in (8, 4, 2, 1) if B % g == 0)

    kfn = partial(_attn_kernel, G=G, S=S, H=H, num_heads=nh, head_dim=hd)

    out = pl.pallas_call(
        kfn,
        out_shape=jax.ShapeDtypeStruct((B, S, H), hidden_states.dtype),
        grid=(B // G,),
        in_specs=[
            pl.BlockSpec((G, S, H), lambda b: (b, 0, 0)),
            pl.BlockSpec(w_pack.shape, lambda b: (0, 0)),
            pl.BlockSpec(vec_pack.shape, lambda b: (0, 0)),
        ],
        out_specs=pl.BlockSpec((G, S, H), lambda b: (b, 0, 0)),
        compiler_params=pltpu.CompilerParams(
            dimension_semantics=("parallel",)),
    )(hidden_states, w_pack, vec_pack)

    return out
